# baseline (speedup 1.0000x reference)
import functools

import jax
import jax.numpy as jnp
from jax import lax
from jax.experimental import pallas as pl
from jax.experimental.pallas import tpu as pltpu
from jax.experimental.pallas import tpu_sc as plsc

_K = 8192
_D = 32
_EPS = 1e-5
_BETA = 0.2
_ROWS = 4096
_BLK = 128


def _conv2d(x, w, b):
    out = lax.conv_general_dilated(
        x, w, window_strides=(2, 2), padding=((1, 1), (1, 1)),
        dimension_numbers=('NCHW', 'OIHW', 'NCHW'))
    return out + b[None, :, None, None]


def _conv_transpose2d(x, w, b):
    w_f = jnp.flip(w, axis=(2, 3))
    w_t = jnp.transpose(w_f, (1, 0, 2, 3))
    out = lax.conv_general_dilated(
        x, w_t, window_strides=(1, 1), padding=((2, 2), (2, 2)),
        lhs_dilation=(2, 2), dimension_numbers=('NCHW', 'OIHW', 'NCHW'))
    return out + b[None, :, None, None]


def _batchnorm(x, g, b):
    m = jnp.mean(x, axis=(0, 2, 3), keepdims=True)
    v = jnp.var(x, axis=(0, 2, 3), keepdims=True)
    xn = (x - m) / jnp.sqrt(v + _EPS)
    return xn * g[None, :, None, None] + b[None, :, None, None]


def _vq_body(q_ref, cb_ref, idx_ref, msum_ref):
    i = pl.program_id(0)
    q = q_ref[...]
    cb = cb_ref[...]
    q2 = jnp.sum(q * q, axis=1, keepdims=True)
    e2 = jnp.sum(cb * cb, axis=1)[None, :]
    cross = lax.dot_general(
        q.astype(jnp.bfloat16), cb.astype(jnp.bfloat16),
        dimension_numbers=(((1,), (1,)), ((), ())),
        preferred_element_type=jnp.float32)
    scores = (q2 + e2) - 2.0 * cross
    m = jnp.min(scores, axis=1, keepdims=True)
    kiota = lax.broadcasted_iota(jnp.int32, scores.shape, 1)
    idx = jnp.min(jnp.where(scores == m, kiota, _K), axis=1)
    idx_ref[0, 0, :] = idx
    blk_sum = jnp.sum(jnp.maximum(m, 0.0))

    @pl.when(i == 0)
    def _():
        msum_ref[0, 0] = 0.0

    msum_ref[0, 0] += blk_sum


def _vq_argmin(q, cb):
    nblk = _ROWS // _BLK
    idx3, msum = pl.pallas_call(
        _vq_body,
        grid=(nblk,),
        in_specs=[
            pl.BlockSpec((_BLK, _D), lambda i: (i, 0)),
            pl.BlockSpec((_K, _D), lambda i: (0, 0)),
        ],
        out_specs=[
            pl.BlockSpec((1, 1, _BLK), lambda i: (i, 0, 0)),
            pl.BlockSpec(memory_space=pltpu.SMEM),
        ],
        out_shape=[
            jax.ShapeDtypeStruct((nblk, 1, _BLK), jnp.int32),
            jax.ShapeDtypeStruct((1, 1), jnp.float32),
        ],
    )(q, cb)
    return idx3.reshape(_ROWS), msum[0, 0]


@functools.lru_cache(maxsize=1)
def _make_sc_gather():
    info = plsc.get_sparse_core_info()
    nw = info.num_cores * info.num_subcores
    b_per_w = _ROWS // nw
    mesh = plsc.VectorSubcoreMesh(core_axis_name="c", subcore_axis_name="s")

    @functools.partial(
        pl.kernel, mesh=mesh,
        out_type=jax.ShapeDtypeStruct((_ROWS, _D), jnp.float32),
        scratch_types=[
            pltpu.VMEM((b_per_w,), jnp.int32),
            pltpu.VMEM((b_per_w, _D), jnp.float32),
            pltpu.SemaphoreType.DMA,
        ],
        compiler_params=pltpu.CompilerParams(use_tc_tiling_on_sc=False),
    )
    def gather(table_hbm, idx_hbm, out_hbm, idx_v, rows_v, sem):
        wid = lax.axis_index("s") * info.num_cores + lax.axis_index("c")
        base = wid * b_per_w
        pltpu.sync_copy(idx_hbm.at[pl.ds(base, b_per_w)], idx_v)
        pltpu.async_copy(table_hbm.at[idx_v], rows_v, sem).wait()
        pltpu.sync_copy(rows_v, out_hbm.at[pl.ds(base, b_per_w)])

    return gather


def _gather_rows(table, idx):
    return _make_sc_gather()(table, idx)


def kernel(x, ew1, eb1, eg1, ebt1, ew2, eb2, eg2, ebt2, ew3, eb3, eg3, ebt3,
           codebook, dw1, db1, dg1, dbt1, dw2, db2, dg2, dbt2, dw3, db3, dg3, dbt3):
    h = jax.nn.relu(_batchnorm(_conv2d(x, ew1, eb1), eg1, ebt1))
    h = jax.nn.relu(_batchnorm(_conv2d(h, ew2, eb2), eg2, ebt2))
    quant_input = jax.nn.relu(_batchnorm(_conv2d(h, ew3, eb3), eg3, ebt3))
    B, C, H, W = quant_input.shape
    q = jnp.transpose(quant_input, (0, 2, 3, 1)).reshape(B * H * W, C)

    idx, msum = _vq_argmin(q, codebook)

    rows = _gather_rows(codebook, idx)

    quant_nchw = jnp.transpose(rows.reshape(B, H, W, C), (0, 3, 1, 2))

    mse = msum / (B * H * W * C)
    qloss = (1.0 + _BETA) * mse

    d = jax.nn.relu(_batchnorm(_conv_transpose2d(quant_nchw, dw1, db1), dg1, dbt1))
    d = jax.nn.relu(_batchnorm(_conv_transpose2d(d, dw2, db2), dg2, dbt2))
    output = _batchnorm(_conv_transpose2d(d, dw3, db3), dg3, dbt3)

    reconstruction_loss = jnp.mean((x - output) ** 2)
    total_loss = qloss + reconstruction_loss
    enc_idx = idx.reshape(B, H, W)
    return (output, total_loss, enc_idx)


# speedup vs baseline: 1.0420x; 1.0420x over previous
import functools

import jax
import jax.numpy as jnp
from jax import lax
from jax.experimental import pallas as pl
from jax.experimental.pallas import tpu as pltpu
from jax.experimental.pallas import tpu_sc as plsc

_K = 8192
_D = 32
_EPS = 1e-5
_BETA = 0.2
_ROWS = 4096
_BLK = 128


def _conv2d(x, w, b):
    out = lax.conv_general_dilated(
        x, w, window_strides=(2, 2), padding=((1, 1), (1, 1)),
        dimension_numbers=('NCHW', 'OIHW', 'NCHW'))
    return out + b[None, :, None, None]


def _conv_transpose2d(x, w, b):
    w_f = jnp.flip(w, axis=(2, 3))
    w_t = jnp.transpose(w_f, (1, 0, 2, 3))
    out = lax.conv_general_dilated(
        x, w_t, window_strides=(1, 1), padding=((2, 2), (2, 2)),
        lhs_dilation=(2, 2), dimension_numbers=('NCHW', 'OIHW', 'NCHW'))
    return out + b[None, :, None, None]


def _batchnorm(x, g, b):
    m = jnp.mean(x, axis=(0, 2, 3), keepdims=True)
    v = jnp.var(x, axis=(0, 2, 3), keepdims=True)
    xn = (x - m) / jnp.sqrt(v + _EPS)
    return xn * g[None, :, None, None] + b[None, :, None, None]


_BLK = 256
_KC = 2048


def _vq_body(q_ref, cb_ref, idx_ref, msum_ref, e2_ref, cbb_ref):
    i = pl.program_id(0)

    @pl.when(i == 0)
    def _():
        cb = cb_ref[...]
        e2_ref[...] = jnp.sum(cb * cb, axis=1)[None, :]
        cbb_ref[...] = cb.astype(jnp.bfloat16)
        msum_ref[0, 0] = 0.0

    q = q_ref[...]
    qb = q.astype(jnp.bfloat16)
    q2 = jnp.sum(q * q, axis=1, keepdims=True)
    run_min = jnp.full((_BLK,), jnp.inf, jnp.float32)
    run_idx = jnp.zeros((_BLK,), jnp.int32)
    for k in range(_K // _KC):
        cb_c = cbb_ref[k * _KC:(k + 1) * _KC, :]
        cross = lax.dot_general(
            qb, cb_c, dimension_numbers=(((1,), (1,)), ((), ())),
            preferred_element_type=jnp.float32)
        scores = (q2 + e2_ref[:, k * _KC:(k + 1) * _KC]) - 2.0 * cross
        m_c = jnp.min(scores, axis=1)
        kio = lax.broadcasted_iota(jnp.int32, scores.shape, 1) + k * _KC
        idx_c = jnp.min(jnp.where(scores == m_c[:, None], kio, _K), axis=1)
        upd = m_c < run_min
        run_idx = jnp.where(upd, idx_c, run_idx)
        run_min = jnp.minimum(run_min, m_c)
    idx_ref[0, 0, :] = run_idx
    msum_ref[0, 0] += jnp.sum(jnp.maximum(run_min, 0.0))


def _vq_argmin(q, cb):
    nblk = _ROWS // _BLK
    idx3, msum = pl.pallas_call(
        _vq_body,
        grid=(nblk,),
        in_specs=[
            pl.BlockSpec((_BLK, _D), lambda i: (i, 0)),
            pl.BlockSpec((_K, _D), lambda i: (0, 0)),
        ],
        out_specs=[
            pl.BlockSpec((1, 1, _BLK), lambda i: (i, 0, 0)),
            pl.BlockSpec(memory_space=pltpu.SMEM),
        ],
        out_shape=[
            jax.ShapeDtypeStruct((nblk, 1, _BLK), jnp.int32),
            jax.ShapeDtypeStruct((1, 1), jnp.float32),
        ],
        scratch_shapes=[
            pltpu.VMEM((1, _K), jnp.float32),
            pltpu.VMEM((_K, _D), jnp.bfloat16),
        ],
    )(q, cb)
    return idx3.reshape(_ROWS), msum[0, 0]


@functools.lru_cache(maxsize=1)
def _make_sc_gather():
    info = plsc.get_sparse_core_info()
    nw = info.num_cores * info.num_subcores
    b_per_w = _ROWS // nw
    mesh = plsc.VectorSubcoreMesh(core_axis_name="c", subcore_axis_name="s")

    @functools.partial(
        pl.kernel, mesh=mesh,
        out_type=jax.ShapeDtypeStruct((_ROWS, _D), jnp.float32),
        scratch_types=[
            pltpu.VMEM((b_per_w,), jnp.int32),
            pltpu.VMEM((b_per_w, _D), jnp.float32),
            pltpu.SemaphoreType.DMA,
        ],
        compiler_params=pltpu.CompilerParams(use_tc_tiling_on_sc=False),
    )
    def gather(table_hbm, idx_hbm, out_hbm, idx_v, rows_v, sem):
        wid = lax.axis_index("s") * info.num_cores + lax.axis_index("c")
        base = wid * b_per_w
        pltpu.sync_copy(idx_hbm.at[pl.ds(base, b_per_w)], idx_v)
        pltpu.async_copy(table_hbm.at[idx_v], rows_v, sem).wait()
        pltpu.sync_copy(rows_v, out_hbm.at[pl.ds(base, b_per_w)])

    return gather


def _gather_rows(table, idx):
    return _make_sc_gather()(table, idx)


def kernel(x, ew1, eb1, eg1, ebt1, ew2, eb2, eg2, ebt2, ew3, eb3, eg3, ebt3,
           codebook, dw1, db1, dg1, dbt1, dw2, db2, dg2, dbt2, dw3, db3, dg3, dbt3):
    h = jax.nn.relu(_batchnorm(_conv2d(x, ew1, eb1), eg1, ebt1))
    h = jax.nn.relu(_batchnorm(_conv2d(h, ew2, eb2), eg2, ebt2))
    quant_input = jax.nn.relu(_batchnorm(_conv2d(h, ew3, eb3), eg3, ebt3))
    B, C, H, W = quant_input.shape
    q = jnp.transpose(quant_input, (0, 2, 3, 1)).reshape(B * H * W, C)

    idx, msum = _vq_argmin(q, codebook)

    rows = _gather_rows(codebook, idx)

    quant_nchw = jnp.transpose(rows.reshape(B, H, W, C), (0, 3, 1, 2))

    mse = msum / (B * H * W * C)
    qloss = (1.0 + _BETA) * mse

    d = jax.nn.relu(_batchnorm(_conv_transpose2d(quant_nchw, dw1, db1), dg1, dbt1))
    d = jax.nn.relu(_batchnorm(_conv_transpose2d(d, dw2, db2), dg2, dbt2))
    output = _batchnorm(_conv_transpose2d(d, dw3, db3), dg3, dbt3)

    reconstruction_loss = jnp.mean((x - output) ** 2)
    total_loss = qloss + reconstruction_loss
    enc_idx = idx.reshape(B, H, W)
    return (output, total_loss, enc_idx)
